# Initial kernel scaffold; baseline (speedup 1.0000x reference)
import jax
import jax.numpy as jnp
from jax.experimental import pallas as pl


def kernel(masked_y_true, K, edge_index):
    raise NotImplementedError("write your pallas kernel here")


# speedup vs baseline: 5.2925x; 5.2925x over previous
import functools

import jax
import jax.numpy as jnp
from jax import lax
from jax.experimental import pallas as pl
from jax.experimental.pallas import tpu as pltpu
from jax.experimental.pallas import tpu_sc as plsc

N = 10000
NP = 10240
E = 320000
C = 128
CH = 64
KHOPS = 3
ALPHA = 0.9

NC = 2
NS = 16
L = 16

RPT = NP // NS
EPT = E // NS
EB = 80
NEB = EPT // EB
RB = 128
NRB = RPT // RB
DW = 16

_mesh = plsc.VectorSubcoreMesh(core_axis_name="c", subcore_axis_name="s")
_sc_params = pltpu.CompilerParams(use_tc_tiling_on_sc=False)


def _zero_vmem(ref, nrows, width):
    @pl.loop(0, nrows)
    def _(i):
        for j in range(width // L):
            ref[i, pl.ds(j * L, L)] = jnp.zeros((L,), jnp.float32)


@functools.partial(
    pl.kernel,
    out_type=jax.ShapeDtypeStruct((NC, NP, DW), jnp.float32),
    mesh=_mesh,
    compiler_params=_sc_params,
    scratch_types=[
        pltpu.VMEM_SHARED((NP, DW), jnp.float32),
        pltpu.VMEM((RPT, DW), jnp.float32),
        pltpu.VMEM((EB, DW), jnp.float32),
        pltpu.VMEM((EB,), jnp.int32),
    ],
)
def _deg_kernel(dst_hbm, deg_out, accdeg, zv, ones_v, dst_v):
    c = lax.axis_index("c")
    s = lax.axis_index("s")

    _zero_vmem(zv, RPT, DW)

    @pl.loop(0, EB)
    def _(i):
        ones_v[i, pl.ds(0, L)] = jnp.ones((L,), jnp.float32)

    pltpu.sync_copy(zv, accdeg.at[pl.ds(s * RPT, RPT)])
    plsc.subcore_barrier()

    @pl.loop(0, NEB)
    def _(b):
        e0 = s * EPT + b * EB
        pltpu.sync_copy(dst_hbm.at[pl.ds(e0, EB)], dst_v)
        pltpu.sync_copy(ones_v, accdeg.at[dst_v], add=True)

    plsc.subcore_barrier()
    pltpu.sync_copy(accdeg.at[pl.ds(s * RPT, RPT)],
                    deg_out.at[c, pl.ds(s * RPT, RPT)])


@functools.partial(
    pl.kernel,
    out_type=jax.ShapeDtypeStruct((NC, NP, CH), jnp.float32),
    mesh=_mesh,
    compiler_params=_sc_params,
    scratch_types=[
        pltpu.VMEM_SHARED((NP, CH), jnp.float32),
        pltpu.VMEM_SHARED((NP, CH), jnp.float32),
        pltpu.VMEM((RB, CH), jnp.float32),
        pltpu.VMEM((EB,), jnp.int32),
        pltpu.VMEM((EB,), jnp.int32),
        pltpu.VMEM((EB, CH), jnp.float32),
        pltpu.VMEM((RB, CH), jnp.float32),
        pltpu.VMEM((RB, CH), jnp.float32),
        pltpu.VMEM((RB, CH), jnp.float32),
        pltpu.VMEM((RB, CH), jnp.float32),
        pltpu.SemaphoreType.DMA,
    ],
)
def _lp3_kernel(y_hbm, dinv_hbm, src_hbm, dst_hbm,
                yout_hbm,
                acc, z_sh, dv, src_v, dst_v, rows_v,
                gv, yv, zv, zero_v, sem):
    c = lax.axis_index("c")
    s = lax.axis_index("s")
    row_base = s * RPT

    _zero_vmem(zero_v, RB, CH)

    for r in range(NRB):
        row0 = row_base + r * RB
        pltpu.sync_copy(y_hbm.at[c, pl.ds(row0, RB)], yv)
        pltpu.sync_copy(dinv_hbm.at[pl.ds(row0, RB)], dv)

        @pl.loop(0, RB)
        def _(i):
            for j in range(CH // L):
                sl = pl.ds(j * L, L)
                zv[i, sl] = dv[i, sl] * yv[i, sl]

        pltpu.sync_copy(zv, z_sh.at[pl.ds(row0, RB)])
        pltpu.sync_copy(zero_v, acc.at[pl.ds(row0, RB)])

    plsc.subcore_barrier()

    for hop in range(KHOPS):
        last = hop == KHOPS - 1

        @pl.loop(0, NEB)
        def _(b):
            e0 = s * EPT + b * EB
            pltpu.sync_copy(src_hbm.at[pl.ds(e0, EB)], src_v)
            pltpu.sync_copy(dst_hbm.at[pl.ds(e0, EB)], dst_v)
            pltpu.async_copy(z_sh.at[src_v], rows_v, sem).wait()
            pltpu.sync_copy(rows_v, acc.at[dst_v], add=True)

        plsc.subcore_barrier()

        ysrc = y_hbm if hop == 0 else yout_hbm
        for r in range(NRB):
            row0 = row_base + r * RB
            pltpu.sync_copy(acc.at[pl.ds(row0, RB)], gv)
            pltpu.sync_copy(ysrc.at[c, pl.ds(row0, RB)], yv)
            pltpu.sync_copy(dinv_hbm.at[pl.ds(row0, RB)], dv)

            @pl.loop(0, RB)
            def _(i):
                for j in range(CH // L):
                    sl = pl.ds(j * L, L)
                    d = dv[i, sl]
                    yn = ALPHA * (d * gv[i, sl]) + (1.0 - ALPHA) * yv[i, sl]
                    yv[i, sl] = yn
                    if not last:
                        zv[i, sl] = d * yn

            pltpu.sync_copy(yv, yout_hbm.at[c, pl.ds(row0, RB)])
            if not last:
                pltpu.sync_copy(zv, z_sh.at[pl.ds(row0, RB)])
                pltpu.sync_copy(zero_v, acc.at[pl.ds(row0, RB)])

        if not last:
            plsc.subcore_barrier()


def _dinv_body(deg_ref, out_ref):
    deg = deg_ref[...][:, 0:1]
    dinv = jnp.where(deg > 0.0, lax.rsqrt(jnp.maximum(deg, 1.0)), 0.0)
    out_ref[...] = jnp.broadcast_to(dinv, (NP, CH))


def _dinv_tc(deg):
    return pl.pallas_call(
        _dinv_body,
        out_shape=jax.ShapeDtypeStruct((NP, CH), jnp.float32),
    )(deg)


BM = 400


def _mm_body(k_ref, y_ref, o_ref):
    o_ref[...] = jnp.dot(k_ref[...], y_ref[...],
                         preferred_element_type=jnp.float32)


def _matmul_tc(kmat, y):
    return pl.pallas_call(
        _mm_body,
        grid=(N // BM,),
        in_specs=[
            pl.BlockSpec((BM, N), lambda i: (i, 0)),
            pl.BlockSpec((N, C), lambda i: (0, 0)),
        ],
        out_specs=pl.BlockSpec((BM, C), lambda i: (i, 0)),
        out_shape=jax.ShapeDtypeStruct((N, C), jnp.float32),
    )(kmat, y)


def _split_pad(y):
    y_split = jnp.stack([y[:, :CH], y[:, CH:]])
    return jnp.pad(y_split, ((0, 0), (0, NP - N), (0, 0)))


def kernel(masked_y_true, K, edge_index):
    src = edge_index[0]
    dst = edge_index[1]

    deg = _deg_kernel(dst)[0]
    dinv = _dinv_tc(deg)

    y3 = _lp3_kernel(_split_pad(masked_y_true), dinv, src, dst)

    y_mid = _matmul_tc(K, jnp.concatenate([y3[0, :N], y3[1, :N]], axis=1))

    y7 = _lp3_kernel(_split_pad(y_mid), dinv, src, dst)

    return jnp.concatenate([y7[0, :N], y7[1, :N]], axis=1)


# speedup vs baseline: 9.7886x; 1.8495x over previous
import functools

import jax
import jax.numpy as jnp
from jax import lax
from jax.experimental import pallas as pl
from jax.experimental.pallas import tpu as pltpu
from jax.experimental.pallas import tpu_sc as plsc

N = 10000
NP = 10240
E = 320000
C = 128
CH = 64
KHOPS = 3
ALPHA = 0.9

NC = 2
NS = 16
L = 16

RPT = NP // NS
EB = 128
EPT = 20480
NEB = EPT // EB
NPAIR = NEB // 2
EPAD = NS * EPT + 2 * EB
RB = 128
NRB = RPT // RB
DW = 16

_mesh = plsc.VectorSubcoreMesh(core_axis_name="c", subcore_axis_name="s")
_sc_params = pltpu.CompilerParams(use_tc_tiling_on_sc=False)


def _zero_vmem(ref, nrows, width):
    @pl.loop(0, nrows)
    def _(i):
        for j in range(width // L):
            ref[i, pl.ds(j * L, L)] = jnp.zeros((L,), jnp.float32)


@functools.partial(
    pl.kernel,
    out_type=jax.ShapeDtypeStruct((NC, NP, DW), jnp.float32),
    mesh=_mesh,
    compiler_params=_sc_params,
    scratch_types=[
        pltpu.VMEM_SHARED((NP, DW), jnp.float32),
        pltpu.VMEM((RPT, DW), jnp.float32),
        pltpu.VMEM((EB, DW), jnp.float32),
        pltpu.VMEM((EB,), jnp.int32),
    ],
)
def _deg_kernel(dst_hbm, deg_out, accdeg, zv, ones_v, dst_v):
    c = lax.axis_index("c")
    s = lax.axis_index("s")

    _zero_vmem(zv, RPT, DW)

    @pl.loop(0, EB)
    def _(i):
        ones_v[i, pl.ds(0, L)] = jnp.ones((L,), jnp.float32)

    pltpu.sync_copy(zv, accdeg.at[pl.ds(s * RPT, RPT)])
    plsc.subcore_barrier()

    @pl.loop(0, NEB)
    def _(b):
        e0 = s * EPT + b * EB
        pltpu.sync_copy(dst_hbm.at[pl.ds(e0, EB)], dst_v)
        pltpu.sync_copy(ones_v, accdeg.at[dst_v], add=True)

    plsc.subcore_barrier()
    pltpu.sync_copy(accdeg.at[pl.ds(s * RPT, RPT)],
                    deg_out.at[c, pl.ds(s * RPT, RPT)])


@functools.partial(
    pl.kernel,
    out_type=jax.ShapeDtypeStruct((NC, NP, CH), jnp.float32),
    mesh=_mesh,
    compiler_params=_sc_params,
    scratch_types=[
        pltpu.VMEM_SHARED((NP, CH), jnp.float32),
        pltpu.VMEM_SHARED((NP, CH), jnp.float32),
        pltpu.VMEM((RB, CH), jnp.float32),
        pltpu.VMEM((EB,), jnp.int32),
        pltpu.VMEM((EB,), jnp.int32),
        pltpu.VMEM((EB,), jnp.int32),
        pltpu.VMEM((EB,), jnp.int32),
        pltpu.VMEM((EB, CH), jnp.float32),
        pltpu.VMEM((EB, CH), jnp.float32),
        pltpu.VMEM((RB, CH), jnp.float32),
        pltpu.VMEM((RB, CH), jnp.float32),
        pltpu.SemaphoreType.DMA,
        pltpu.SemaphoreType.DMA,
        pltpu.SemaphoreType.DMA,
        pltpu.SemaphoreType.DMA,
    ],
)
def _lp3_kernel(y_hbm, dinv_hbm, src_hbm, dst_hbm,
                yout_hbm,
                acc, z_sh, dv, sv0, dv0, sv1, dv1, rb0, rb1,
                gv, yv, gsem0, gsem1, ssem0, ssem1):
    c = lax.axis_index("c")
    s = lax.axis_index("s")
    row_base = s * RPT

    def drain(sem, buf):
        pltpu.make_async_copy(y_hbm.at[c, pl.ds(0, EB)], buf, sem).wait()

    def zero_gv():
        @pl.loop(0, RB)
        def _(i):
            for j in range(CH // L):
                gv[i, pl.ds(j * L, L)] = jnp.zeros((L,), jnp.float32)

    for r in range(NRB):
        row0 = row_base + r * RB
        pltpu.sync_copy(y_hbm.at[c, pl.ds(row0, RB)], yv)
        pltpu.sync_copy(dinv_hbm.at[pl.ds(row0, RB)], dv)

        @pl.loop(0, RB)
        def _(i):
            for j in range(CH // L):
                sl = pl.ds(j * L, L)
                gv[i, sl] = dv[i, sl] * yv[i, sl]

        pltpu.sync_copy(gv, z_sh.at[pl.ds(row0, RB)])
        zero_gv()
        pltpu.sync_copy(gv, acc.at[pl.ds(row0, RB)])

    plsc.subcore_barrier()

    for hop in range(KHOPS):
        last = hop == KHOPS - 1
        ebase = s * EPT

        pltpu.sync_copy(src_hbm.at[pl.ds(ebase, EB)], sv0)
        pltpu.sync_copy(dst_hbm.at[pl.ds(ebase, EB)], dv0)
        pltpu.async_copy(z_sh.at[sv0], rb0, gsem0)
        pltpu.async_copy(y_hbm.at[c, pl.ds(0, EB)], rb1, ssem1)

        @pl.loop(0, NPAIR)
        def _(p):
            e0 = ebase + 2 * p * EB
            pltpu.sync_copy(src_hbm.at[pl.ds(e0 + EB, EB)], sv1)
            pltpu.sync_copy(dst_hbm.at[pl.ds(e0 + EB, EB)], dv1)
            drain(ssem1, rb1)
            pltpu.async_copy(z_sh.at[sv1], rb1, gsem1)
            drain(gsem0, rb0)
            pltpu.async_copy(rb0, acc.at[dv0], ssem0, add=True)
            pltpu.sync_copy(src_hbm.at[pl.ds(e0 + 2 * EB, EB)], sv0)
            pltpu.sync_copy(dst_hbm.at[pl.ds(e0 + 2 * EB, EB)], dv0)
            drain(ssem0, rb0)
            pltpu.async_copy(z_sh.at[sv0], rb0, gsem0)
            drain(gsem1, rb1)
            pltpu.async_copy(rb1, acc.at[dv1], ssem1, add=True)

        drain(ssem1, rb1)
        drain(gsem0, rb0)

        plsc.subcore_barrier()

        ysrc = y_hbm if hop == 0 else yout_hbm
        for r in range(NRB):
            row0 = row_base + r * RB
            pltpu.sync_copy(acc.at[pl.ds(row0, RB)], gv)
            pltpu.sync_copy(ysrc.at[c, pl.ds(row0, RB)], yv)
            pltpu.sync_copy(dinv_hbm.at[pl.ds(row0, RB)], dv)

            @pl.loop(0, RB)
            def _(i):
                for j in range(CH // L):
                    sl = pl.ds(j * L, L)
                    d = dv[i, sl]
                    yn = ALPHA * (d * gv[i, sl]) + (1.0 - ALPHA) * yv[i, sl]
                    yv[i, sl] = yn
                    if not last:
                        gv[i, sl] = d * yn

            pltpu.sync_copy(yv, yout_hbm.at[c, pl.ds(row0, RB)])
            if not last:
                pltpu.sync_copy(gv, z_sh.at[pl.ds(row0, RB)])
                zero_gv()
                pltpu.sync_copy(gv, acc.at[pl.ds(row0, RB)])

        if not last:
            plsc.subcore_barrier()


def _dinv_body(deg_ref, out_ref):
    deg = deg_ref[...][:, 0:1]
    dinv = jnp.where(deg > 0.0, lax.rsqrt(jnp.maximum(deg, 1.0)), 0.0)
    out_ref[...] = jnp.broadcast_to(dinv, (NP, CH))


def _dinv_tc(deg):
    return pl.pallas_call(
        _dinv_body,
        out_shape=jax.ShapeDtypeStruct((NP, CH), jnp.float32),
    )(deg)


BM = 400


def _mm_body(k_ref, y_ref, o_ref):
    o_ref[...] = jnp.dot(k_ref[...], y_ref[...],
                         preferred_element_type=jnp.float32)


def _matmul_tc(kmat, y):
    return pl.pallas_call(
        _mm_body,
        grid=(N // BM,),
        in_specs=[
            pl.BlockSpec((BM, N), lambda i: (i, 0)),
            pl.BlockSpec((N, C), lambda i: (0, 0)),
        ],
        out_specs=pl.BlockSpec((BM, C), lambda i: (i, 0)),
        out_shape=jax.ShapeDtypeStruct((N, C), jnp.float32),
    )(kmat, y)


def _split_pad(y):
    y_split = jnp.stack([y[:, :CH], y[:, CH:]])
    return jnp.pad(y_split, ((0, 0), (0, NP - N), (0, 0)))


def kernel(masked_y_true, K, edge_index):
    pad = jnp.full((2, EPAD - E), NP - 1, dtype=jnp.int32)
    ei = jnp.concatenate([edge_index, pad], axis=1)
    src = ei[0]
    dst = ei[1]

    deg = _deg_kernel(dst)[0]
    dinv = _dinv_tc(deg)

    y3 = _lp3_kernel(_split_pad(masked_y_true), dinv, src, dst)

    y_mid = _matmul_tc(K, jnp.concatenate([y3[0, :N], y3[1, :N]], axis=1))

    y7 = _lp3_kernel(_split_pad(y_mid), dinv, src, dst)

    return jnp.concatenate([y7[0, :N], y7[1, :N]], axis=1)
